# baseline (speedup 1.0000x reference)
import functools

import jax
import jax.numpy as jnp
from jax import lax
from jax.experimental import pallas as pl
from jax.experimental.pallas import tpu as pltpu
from jax.experimental.pallas import tpu_sc as plsc

HIDDEN = 64
L = 16
NC, NS = 2, 16
NW = NC * NS
CHUNK = 128


@functools.partial(jax.jit, static_argnames=("B",))
def _run(user, item, user_factors, item_factors, user_biases, item_biases, B):
    b_per_w = B // NW
    n_chunks = b_per_w // CHUNK
    mesh = plsc.VectorSubcoreMesh(core_axis_name="c", subcore_axis_name="s")

    @functools.partial(
        pl.kernel,
        mesh=mesh,
        compiler_params=pltpu.CompilerParams(
            needs_layout_passes=False, use_tc_tiling_on_sc=False),
        out_type=jax.ShapeDtypeStruct((B,), jnp.float32),
        scratch_types=[
            pltpu.VMEM((CHUNK,), jnp.int32),
            pltpu.VMEM((CHUNK,), jnp.int32),
            pltpu.VMEM((CHUNK, HIDDEN), jnp.float32),
            pltpu.VMEM((CHUNK, HIDDEN), jnp.float32),
            pltpu.VMEM((CHUNK, HIDDEN), jnp.float32),
            pltpu.VMEM((CHUNK, HIDDEN), jnp.float32),
            pltpu.VMEM((CHUNK,), jnp.float32),
            pltpu.SemaphoreType.DMA,
        ],
    )
    def k(user_hbm, item_hbm, uf_hbm, if_hbm, ub_hbm, ib_hbm, out_hbm,
          uidx_v, iidx_v, uf_v, if_v, ub_v, ib_v, o_v, sem):
        wid = lax.axis_index("s") * NC + lax.axis_index("c")
        base = wid * b_per_w
        for c in range(n_chunks):
            off = base + c * CHUNK
            pltpu.sync_copy(user_hbm.at[pl.ds(off, CHUNK)], uidx_v)
            pltpu.sync_copy(item_hbm.at[pl.ds(off, CHUNK)], iidx_v)
            cps = [
                pltpu.async_copy(uf_hbm.at[uidx_v], uf_v, sem),
                pltpu.async_copy(if_hbm.at[iidx_v], if_v, sem),
                pltpu.async_copy(ub_hbm.at[uidx_v], ub_v, sem),
                pltpu.async_copy(ib_hbm.at[iidx_v], ib_v, sem),
            ]
            for cp in cps:
                cp.wait()

            lane = lax.iota(jnp.int32, L)

            def group(g, _):
                vec = jnp.zeros((L,), jnp.float32)
                for j in range(L):
                    r = g * L + j
                    acc = jnp.zeros((L,), jnp.float32)
                    for kk in range(HIDDEN // L):
                        sl = pl.ds(kk * L, L)
                        acc = acc + (uf_v[r, sl] * if_v[r, sl]
                                     + ub_v[r, sl] + ib_v[r, sl])
                    vec = jnp.where(lane == j, jnp.sum(acc), vec)
                o_v[pl.ds(g * L, L)] = vec
                return 0

            lax.fori_loop(0, CHUNK // L, group, 0)
            pltpu.sync_copy(o_v, out_hbm.at[pl.ds(off, CHUNK)])

    return k(user.astype(jnp.int32), item.astype(jnp.int32),
             user_factors, item_factors, user_biases, item_biases)


def kernel(user, item, user_factors, item_factors, user_biases, item_biases):
    B = user.shape[0]
    out = _run(user, item, user_factors, item_factors, user_biases,
               item_biases, B)
    return out.reshape(B, 1)


# speedup vs baseline: 1.4646x; 1.4646x over previous
import functools

import jax
import jax.numpy as jnp
from jax import lax
from jax.experimental import pallas as pl
from jax.experimental.pallas import tpu as pltpu
from jax.experimental.pallas import tpu_sc as plsc

HIDDEN = 64
L = 16
NC, NS = 2, 16
NW = NC * NS
CHUNK = 16


@functools.partial(jax.jit, static_argnames=("B",))
def _run(user, item, user_factors, item_factors, user_biases, item_biases, B):
    b_per_w = B // NW
    n_chunks = b_per_w // CHUNK
    mesh = plsc.VectorSubcoreMesh(core_axis_name="c", subcore_axis_name="s")

    @functools.partial(
        pl.kernel,
        mesh=mesh,
        compiler_params=pltpu.CompilerParams(
            needs_layout_passes=False, use_tc_tiling_on_sc=True),
        out_type=jax.ShapeDtypeStruct((B,), jnp.float32),
        scratch_types=[
            pltpu.VMEM((CHUNK,), jnp.int32),
            pltpu.VMEM((CHUNK,), jnp.int32),
            pltpu.VMEM((CHUNK, HIDDEN), jnp.float32),
            pltpu.VMEM((CHUNK, HIDDEN), jnp.float32),
            pltpu.VMEM((CHUNK, HIDDEN), jnp.float32),
            pltpu.VMEM((CHUNK, HIDDEN), jnp.float32),
            pltpu.VMEM((CHUNK,), jnp.float32),
            pltpu.SemaphoreType.DMA,
        ],
    )
    def k(user_hbm, item_hbm, uf_hbm, if_hbm, ub_hbm, ib_hbm, out_hbm,
          uidx_v, iidx_v, uf_v, if_v, ub_v, ib_v, o_v, sem):
        wid = lax.axis_index("s") * NC + lax.axis_index("c")
        base = wid * b_per_w
        lane = lax.iota(jnp.int32, L)

        def chunk_body(c, _):
            off = base + c * CHUNK
            pltpu.sync_copy(user_hbm.at[pl.ds(off, CHUNK)], uidx_v)
            pltpu.sync_copy(item_hbm.at[pl.ds(off, CHUNK)], iidx_v)
            uvec = uidx_v[...]
            ivec = iidx_v[...]
            cps = []
            for j in range(CHUNK):
                ru = uvec[j]
                ri = ivec[j]
                dst = pl.ds(j, 1)
                cps.append(pltpu.async_copy(
                    uf_hbm.at[pl.ds(ru, 1), :], uf_v.at[dst, :], sem))
                cps.append(pltpu.async_copy(
                    if_hbm.at[pl.ds(ri, 1), :], if_v.at[dst, :], sem))
                cps.append(pltpu.async_copy(
                    ub_hbm.at[pl.ds(ru, 1), :], ub_v.at[dst, :], sem))
                cps.append(pltpu.async_copy(
                    ib_hbm.at[pl.ds(ri, 1), :], ib_v.at[dst, :], sem))
            for cp in cps:
                cp.wait()

            vec = jnp.zeros((L,), jnp.float32)
            for j in range(CHUNK):
                acc = jnp.zeros((L,), jnp.float32)
                for kk in range(HIDDEN // L):
                    sl = pl.ds(kk * L, L)
                    acc = acc + (uf_v[j, sl] * if_v[j, sl]
                                 + ub_v[j, sl] + ib_v[j, sl])
                vec = jnp.where(lane == j, jnp.sum(acc), vec)
            o_v[...] = vec
            pltpu.sync_copy(o_v, out_hbm.at[pl.ds(off, CHUNK)])
            return 0

        lax.fori_loop(0, n_chunks, chunk_body, 0)

    return k(user.astype(jnp.int32), item.astype(jnp.int32),
             user_factors, item_factors, user_biases, item_biases)


def kernel(user, item, user_factors, item_factors, user_biases, item_biases):
    B = user.shape[0]
    out = _run(user, item, user_factors, item_factors, user_biases,
               item_biases, B)
    return out.reshape(B, 1)


# speedup vs baseline: 1.4668x; 1.0015x over previous
import functools

import jax
import jax.numpy as jnp
from jax import lax
from jax.experimental import pallas as pl
from jax.experimental.pallas import tpu as pltpu
from jax.experimental.pallas import tpu_sc as plsc

HIDDEN = 64
L = 16
NC, NS = 2, 16
NW = NC * NS
CHUNK = 16


@functools.partial(jax.jit, static_argnames=("B",))
def _run(user, item, user_factors, item_factors, user_biases, item_biases, B):
    b_per_w = B // NW
    n_chunks = b_per_w // CHUNK
    mesh = plsc.VectorSubcoreMesh(core_axis_name="c", subcore_axis_name="s")

    @functools.partial(
        pl.kernel,
        mesh=mesh,
        compiler_params=pltpu.CompilerParams(
            needs_layout_passes=False, use_tc_tiling_on_sc=True,
            skip_device_barrier=True, disable_bounds_checks=True,
            disable_semaphore_checks=True),
        out_type=jax.ShapeDtypeStruct((B,), jnp.float32),
        scratch_types=[
            pltpu.VMEM((CHUNK,), jnp.int32),
            pltpu.VMEM((CHUNK,), jnp.int32),
            pltpu.VMEM((CHUNK, HIDDEN), jnp.float32),
            pltpu.VMEM((CHUNK, HIDDEN), jnp.float32),
            pltpu.VMEM((CHUNK, HIDDEN), jnp.float32),
            pltpu.VMEM((CHUNK, HIDDEN), jnp.float32),
            pltpu.VMEM((CHUNK,), jnp.float32),
            pltpu.SemaphoreType.DMA,
        ],
    )
    def k(user_hbm, item_hbm, uf_hbm, if_hbm, ub_hbm, ib_hbm, out_hbm,
          uidx_v, iidx_v, uf_v, if_v, ub_v, ib_v, o_v, sem):
        wid = lax.axis_index("s") * NC + lax.axis_index("c")
        base = wid * b_per_w
        lane = lax.iota(jnp.int32, L)

        def chunk_body(c, _):
            off = base + c * CHUNK
            pltpu.sync_copy(user_hbm.at[pl.ds(off, CHUNK)], uidx_v)
            pltpu.sync_copy(item_hbm.at[pl.ds(off, CHUNK)], iidx_v)
            uvec = uidx_v[...]
            ivec = iidx_v[...]
            cps = []
            for j in range(CHUNK):
                ru = uvec[j]
                ri = ivec[j]
                dst = pl.ds(j, 1)
                cps.append(pltpu.async_copy(
                    uf_hbm.at[pl.ds(ru, 1), :], uf_v.at[dst, :], sem))
                cps.append(pltpu.async_copy(
                    if_hbm.at[pl.ds(ri, 1), :], if_v.at[dst, :], sem))
                cps.append(pltpu.async_copy(
                    ub_hbm.at[pl.ds(ru, 1), :], ub_v.at[dst, :], sem))
                cps.append(pltpu.async_copy(
                    ib_hbm.at[pl.ds(ri, 1), :], ib_v.at[dst, :], sem))
            for cp in cps:
                cp.wait()

            vec = jnp.zeros((L,), jnp.float32)
            for j in range(CHUNK):
                acc = jnp.zeros((L,), jnp.float32)
                for kk in range(HIDDEN // L):
                    sl = pl.ds(kk * L, L)
                    acc = acc + (uf_v[j, sl] * if_v[j, sl]
                                 + ub_v[j, sl] + ib_v[j, sl])
                vec = jnp.where(lane == j, jnp.sum(acc), vec)
            o_v[...] = vec
            pltpu.sync_copy(o_v, out_hbm.at[pl.ds(off, CHUNK)])
            return 0

        lax.fori_loop(0, n_chunks, chunk_body, 0)

    return k(user.astype(jnp.int32), item.astype(jnp.int32),
             user_factors, item_factors, user_biases, item_biases)


def kernel(user, item, user_factors, item_factors, user_biases, item_biases):
    B = user.shape[0]
    out = _run(user, item, user_factors, item_factors, user_biases,
               item_biases, B)
    return out.reshape(B, 1)
